# baseline (speedup 1.0000x reference)
import functools

import jax
import jax.numpy as jnp
from jax import lax
from jax.experimental import pallas as pl
from jax.experimental.pallas import tpu as pltpu
from jax.experimental.pallas import tpu_sc as plsc

NC = 2
NS = 16
NW = NC * NS

V = 1_000_000
D = 64
NB = 16384
NA = 26
NQ = V // 2

FULL_T = V // 128
TAIL_ROWS = V - FULL_T * 128
K1_ITERS = (FULL_T + NW - 1) // NW

OUT_BLOCKS = NA * NB // 128
K2_ITERS = OUT_BLOCKS // NW

NBUF = 2

_mesh = plsc.VectorSubcoreMesh(
    core_axis_name="c", subcore_axis_name="s", num_cores=NC, num_subcores=NS
)


def _wid():
    return lax.axis_index("s") * NC + lax.axis_index("c")


_VM = pltpu.VMEM
_SEM = pltpu.SemaphoreType.DMA


SB = FULL_T // 4
SB_ITERS = (SB + NW - 1) // NW


@functools.partial(
    pl.kernel,
    out_type=jax.ShapeDtypeStruct((NQ, 128), jnp.float32),
    mesh=_mesh,
    scratch_types=(
        [_VM((64, 513), jnp.float32)] * 2
        + [_VM((128, 128), jnp.float32)] * 2
        + [_SEM] * 4
    ),
    compiler_params=pltpu.CompilerParams(needs_layout_passes=False),
)
def _pack_kernel(tT_hbm, tail_hbm, s_hbm, *refs):
    bufs = refs[0:2]
    sbufs = refs[2:4]
    insems = refs[4:6]
    osems = refs[6:8]
    w = _wid()
    iota = lax.iota(jnp.int32, 16)
    cvecs = [(g % 4) * 16 + iota for g in range(8)]

    def issue_in(kb, p):
        sb = kb * NW + w

        @pl.when((kb < SB_ITERS) & (sb < SB))
        def _():
            pltpu.async_copy(
                tT_hbm.at[:, pl.ds(sb * 512, 512)], bufs[p].at[:, :512], insems[p]
            )

    def step(kb, p):
        sb = kb * NW + w
        issue_in(kb + 1, 1 - p)

        @pl.when((kb < SB_ITERS) & (sb < SB))
        def _():
            pltpu.make_async_copy(
                tT_hbm.at[:, pl.ds(0, 512)], bufs[p].at[:, :512], insems[p]
            ).wait()
            buf = bufs[p]

            for u in range(2):
                @pl.when(kb >= 1)
                def _w():
                    pltpu.make_async_copy(
                        sbufs[u], s_hbm.at[pl.ds(0, 128)], osems[u]
                    ).wait()

                sbuf = sbufs[u]

                @plsc.parallel_loop(0, 128, unroll=2)
                def _sh(m):
                    base = ((m >> 6) + 2 * u) * 128 + 2 * (m & 63)
                    for g in range(8):
                        lvec = jnp.full((16,), base + g // 4, dtype=jnp.int32)
                        sbuf[m, pl.ds(g * 16, 16)] = plsc.load_gather(
                            buf, [cvecs[g], lvec]
                        )
                pltpu.async_copy(
                    sbuf,
                    s_hbm.at[pl.ds(sb * 256 + u * 128, 128)],
                    osems[u],
                )

    issue_in(0, 0)

    def body(j, carry):
        step(2 * j, 0)
        step(2 * j + 1, 1)
        return carry

    lax.fori_loop(0, (SB_ITERS + 1) // 2, body, 0)

    for u in range(2):
        pltpu.make_async_copy(
            sbufs[u], s_hbm.at[pl.ds(0, 128)], osems[u]
        ).wait()

    @pl.when(w == NW - 1)
    def _tail():
        pltpu.sync_copy(tail_hbm, sbufs[0].at[:32])
        pltpu.sync_copy(sbufs[0].at[:32], s_hbm.at[pl.ds(FULL_T * 64, 32)])


@functools.partial(
    pl.kernel,
    out_type=jax.ShapeDtypeStruct((NA, D, NB), jnp.float32),
    mesh=_mesh,
    scratch_types=(
        [_VM((128,), jnp.int32)] * NBUF
        + [_VM((128,), jnp.int32)] * NBUF
        + [_VM((128, 129), jnp.float32)] * NBUF
        + [_VM((64, 128), jnp.float32)] * NBUF
        + [_SEM] * (3 * NBUF)
    ),
    compiler_params=pltpu.CompilerParams(needs_layout_passes=False),
)
def _gather_kernel(iT_hbm, s_hbm, out_hbm, *refs):
    idxbs = refs[0:NBUF]
    qbs = refs[NBUF:2 * NBUF]
    g2ds = refs[2 * NBUF:3 * NBUF]
    obs = refs[3 * NBUF:4 * NBUF]
    isems = refs[4 * NBUF:5 * NBUF]
    gsems = refs[5 * NBUF:6 * NBUF]
    osems = refs[6 * NBUF:7 * NBUF]
    w = _wid()
    iota = lax.iota(jnp.int32, 16)
    rowvecs = [g * 16 + iota for g in range(8)]

    def blk_addr(kb):
        blk = kb * NW + w
        return blk // 128, (blk % 128) * 128

    def issue_idx(kb, p):
        @pl.when(kb < K2_ITERS)
        def _():
            a, b0 = blk_addr(kb)
            pltpu.async_copy(iT_hbm.at[a, pl.ds(b0, 128)], idxbs[p], isems[p])

    def launch_gather(kb, p):
        @pl.when(kb < K2_ITERS)
        def _():
            pltpu.make_async_copy(
                iT_hbm.at[0, pl.ds(0, 128)], idxbs[p], isems[p]
            ).wait()
            for g in range(8):
                qbs[p][pl.ds(g * 16, 16)] = jnp.right_shift(
                    idxbs[p][pl.ds(g * 16, 16)], 1
                )
            pltpu.async_copy(s_hbm.at[qbs[p]], g2ds[p].at[:, :128], gsems[p])

    def step(kb, p):
        a, b0 = blk_addr(kb)
        pltpu.make_async_copy(
            s_hbm.at[qbs[p]], g2ds[p].at[:, :128], gsems[p]
        ).wait()
        launch_gather(kb + NBUF - 1, (p + NBUF - 1) % NBUF)

        g2d, ob, idxb = g2ds[p], obs[p], idxbs[p]
        hoffs = [
            jnp.left_shift(jnp.bitwise_and(idxb[pl.ds(g * 16, 16)], 1), 6)
            for g in range(8)
        ]

        @pl.when(kb >= NBUF)
        def _w():
            pltpu.make_async_copy(
                obs[p], out_hbm.at[0, :, pl.ds(0, 128)], osems[p]
            ).wait()

        @plsc.parallel_loop(0, 64, unroll=2)
        def _sh(c):
            for g in range(8):
                ob[c, pl.ds(g * 16, 16)] = plsc.load_gather(
                    g2d, [rowvecs[g], hoffs[g] + c]
                )
        pltpu.async_copy(ob, out_hbm.at[a, :, pl.ds(b0, 128)], osems[p])
        issue_idx(kb + NBUF, p)

    for d in range(NBUF):
        issue_idx(d, d)
    for d in range(NBUF - 1):
        launch_gather(d, d)

    def body(j, carry):
        for p in range(NBUF):
            step(NBUF * j + p, p)
        return carry

    lax.fori_loop(0, K2_ITERS // NBUF, body, 0)

    for p in range(NBUF):
        pltpu.make_async_copy(
            obs[p], out_hbm.at[0, :, pl.ds(0, 128)], osems[p]
        ).wait()


def kernel(i, types):
    tail = types[FULL_T * 128:].reshape(32, 128)
    s = _pack_kernel(types.T, tail)
    o3 = _gather_kernel(i.T, s)
    return o3.transpose(2, 0, 1)


# speedup vs baseline: 1.2070x; 1.2070x over previous
import functools

import jax
import jax.numpy as jnp
from jax import lax
from jax.experimental import pallas as pl
from jax.experimental.pallas import tpu as pltpu
from jax.experimental.pallas import tpu_sc as plsc

NC = 2
NS = 16
NW = NC * NS

V = 1_000_000
D = 64
NB = 16384
NA = 26
NQ = V // 2

FULL_T = V // 128
TAIL_ROWS = V - FULL_T * 128
K1_ITERS = (FULL_T + NW - 1) // NW

OUT_BLOCKS = NA * NB // 128
K2_ITERS = OUT_BLOCKS // NW

NBUF = 2

_mesh = plsc.VectorSubcoreMesh(
    core_axis_name="c", subcore_axis_name="s", num_cores=NC, num_subcores=NS
)


def _wid():
    return lax.axis_index("s") * NC + lax.axis_index("c")


_VM = pltpu.VMEM
_SEM = pltpu.SemaphoreType.DMA


SB = FULL_T // 4
SB_ITERS = (SB + NW - 1) // NW


@functools.partial(
    pl.kernel,
    out_type=jax.ShapeDtypeStruct((NQ, 128), jnp.float32),
    mesh=_mesh,
    scratch_types=(
        [_VM((64, 513), jnp.float32)] * 2
        + [_VM((128, 128), jnp.float32)] * 2
        + [_SEM] * 4
    ),
    compiler_params=pltpu.CompilerParams(needs_layout_passes=False),
)
def _pack_kernel(tT_hbm, tail_hbm, s_hbm, *refs):
    bufs = refs[0:2]
    sbufs = refs[2:4]
    insems = refs[4:6]
    osems = refs[6:8]
    w = _wid()
    iota = lax.iota(jnp.int32, 16)
    cvecs = [(g % 4) * 16 + iota for g in range(8)]

    def issue_in(kb, p):
        sb = kb * NW + w

        @pl.when((kb < SB_ITERS) & (sb < SB))
        def _():
            pltpu.async_copy(
                tT_hbm.at[:, pl.ds(sb * 512, 512)], bufs[p].at[:, :512], insems[p]
            )

    def step(kb, p):
        sb = kb * NW + w
        issue_in(kb + 1, 1 - p)

        @pl.when((kb < SB_ITERS) & (sb < SB))
        def _():
            pltpu.make_async_copy(
                tT_hbm.at[:, pl.ds(0, 512)], bufs[p].at[:, :512], insems[p]
            ).wait()
            buf = bufs[p]

            for u in range(2):
                @pl.when(kb >= 1)
                def _w():
                    pltpu.make_async_copy(
                        sbufs[u], s_hbm.at[pl.ds(0, 128)], osems[u]
                    ).wait()

                sbuf = sbufs[u]

                @plsc.parallel_loop(0, 128, unroll=2)
                def _sh(m):
                    base = ((m >> 6) + 2 * u) * 128 + 2 * (m & 63)
                    for g in range(8):
                        lvec = jnp.full((16,), base + g // 4, dtype=jnp.int32)
                        sbuf[m, pl.ds(g * 16, 16)] = plsc.load_gather(
                            buf, [cvecs[g], lvec]
                        )
                pltpu.async_copy(
                    sbuf,
                    s_hbm.at[pl.ds(sb * 256 + u * 128, 128)],
                    osems[u],
                )

    issue_in(0, 0)

    def body(j, carry):
        step(2 * j, 0)
        step(2 * j + 1, 1)
        return carry

    lax.fori_loop(0, (SB_ITERS + 1) // 2, body, 0)

    for u in range(2):
        pltpu.make_async_copy(
            sbufs[u], s_hbm.at[pl.ds(0, 128)], osems[u]
        ).wait()

    @pl.when(w == NW - 1)
    def _tail():
        pltpu.sync_copy(tail_hbm, sbufs[0].at[:32])
        pltpu.sync_copy(sbufs[0].at[:32], s_hbm.at[pl.ds(FULL_T * 64, 32)])


@functools.partial(
    pl.kernel,
    out_type=jax.ShapeDtypeStruct((NA, D, NB), jnp.float32),
    mesh=_mesh,
    scratch_types=(
        [_VM((128,), jnp.int32)] * NBUF
        + [_VM((128,), jnp.int32)] * NBUF
        + [_VM((128, 129), jnp.float32)] * NBUF
        + [_VM((64, 128), jnp.float32)] * NBUF
        + [_SEM] * (3 * NBUF)
    ),
    compiler_params=pltpu.CompilerParams(needs_layout_passes=False),
)
def _gather_kernel(iT_hbm, s_hbm, out_hbm, *refs):
    idxbs = refs[0:NBUF]
    qbs = refs[NBUF:2 * NBUF]
    g2ds = refs[2 * NBUF:3 * NBUF]
    obs = refs[3 * NBUF:4 * NBUF]
    isems = refs[4 * NBUF:5 * NBUF]
    gsems = refs[5 * NBUF:6 * NBUF]
    osems = refs[6 * NBUF:7 * NBUF]
    w = _wid()
    iota = lax.iota(jnp.int32, 16)
    rowvecs = [g * 16 + iota for g in range(8)]

    def blk_addr(kb):
        blk = kb * NW + w
        return blk // 128, (blk % 128) * 128

    def issue_idx(kb, p):
        @pl.when(kb < K2_ITERS)
        def _():
            a, b0 = blk_addr(kb)
            pltpu.async_copy(iT_hbm.at[a, pl.ds(b0, 128)], idxbs[p], isems[p])

    def launch_gather(kb, p):
        @pl.when(kb < K2_ITERS)
        def _():
            pltpu.make_async_copy(
                iT_hbm.at[0, pl.ds(0, 128)], idxbs[p], isems[p]
            ).wait()
            for g in range(8):
                qbs[p][pl.ds(g * 16, 16)] = jnp.right_shift(
                    idxbs[p][pl.ds(g * 16, 16)], 1
                )
            pltpu.async_copy(s_hbm.at[qbs[p]], g2ds[p].at[:, :128], gsems[p])

    def step(kb, p):
        a, b0 = blk_addr(kb)
        pltpu.make_async_copy(
            s_hbm.at[qbs[p]], g2ds[p].at[:, :128], gsems[p]
        ).wait()
        launch_gather(kb + NBUF - 1, (p + NBUF - 1) % NBUF)

        g2d, ob, idxb = g2ds[p], obs[p], idxbs[p]
        hoffs = [
            jnp.left_shift(jnp.bitwise_and(idxb[pl.ds(g * 16, 16)], 1), 6)
            for g in range(8)
        ]

        @pl.when(kb >= NBUF)
        def _w():
            pltpu.make_async_copy(
                obs[p], out_hbm.at[0, :, pl.ds(0, 128)], osems[p]
            ).wait()

        @plsc.parallel_loop(0, 64, unroll=2)
        def _sh(c):
            for g in range(8):
                ob[c, pl.ds(g * 16, 16)] = plsc.load_gather(
                    g2d, [rowvecs[g], hoffs[g] + c]
                )
        pltpu.async_copy(ob, out_hbm.at[a, :, pl.ds(b0, 128)], osems[p])
        issue_idx(kb + NBUF, p)

    for d in range(NBUF):
        issue_idx(d, d)
    for d in range(NBUF - 1):
        launch_gather(d, d)

    def body(j, carry):
        for p in range(NBUF):
            step(NBUF * j + p, p)
        return carry

    lax.fori_loop(0, K2_ITERS // NBUF, body, 0)

    for p in range(NBUF):
        pltpu.make_async_copy(
            obs[p], out_hbm.at[0, :, pl.ds(0, 128)], osems[p]
        ).wait()


def kernel(i, types):
    s = jnp.reshape(types, (NQ, 128))
    o3 = _gather_kernel(i.T, s)
    return o3.transpose(2, 0, 1)
